# baseline (speedup 1.0000x reference)
import functools

import numpy as np
import jax
import jax.numpy as jnp
from jax import lax
from jax.experimental import pallas as pl
from jax.experimental.pallas import tpu as pltpu
from jax.experimental.pallas import tpu_sc as plsc

_FREQ_MASK_PARAM = 27
_TIME_MASK_PARAM = 100
_NUM_FREQ_MASKS = 2
_NUM_TIME_MASKS = 2

_B, _F, _T = 64, 128, 3000
_TP = 3008
_LPG = 8
_NG = _F // _LPG
_NW = 32
_ROWS_PER_W = _B // _NW
_LANES = 16
_UNROLL = 4
_NV = _TP // _LANES


def _mask_consts(B, F, T):
    rng = np.random.RandomState(0)
    partner = np.empty(B, dtype=np.int64)
    for i in range(B):
        j = int(rng.randint(0, B - 1))
        if j >= i:
            j += 1
        partner[i] = j
    fmask = np.zeros((B, F), dtype=bool)
    tmask = np.zeros((B, T), dtype=bool)
    for i in range(B):
        for _ in range(_NUM_FREQ_MASKS):
            f = int(rng.randint(0, _FREQ_MASK_PARAM + 1))
            if f == 0:
                continue
            f0 = int(rng.randint(0, F - f + 1))
            fmask[i, f0:f0 + f] = True
        for _ in range(_NUM_TIME_MASKS):
            t = int(rng.randint(0, _TIME_MASK_PARAM + 1))
            if t == 0:
                continue
            t0 = int(rng.randint(0, T - t + 1))
            tmask[i, t0:t0 + t] = True
    return partner, fmask, tmask


_PARTNER, _FMASK, _TMASK = _mask_consts(_B, _F, _T)
_P_LIST = [int(v) for v in _PARTNER]


def _sc_body(x_hbm, fm_hbm, tm_hbm, out_hbm, xbuf, ybuf, tmbuf, fmbuf, sem, osem):
    nc = 2
    wid = lax.axis_index("s") * nc + lax.axis_index("c")

    for r in range(_ROWS_PER_W):
        i = wid * _ROWS_PER_W + r
        p = jnp.int32(_P_LIST[r])
        for w in range(1, _NW):
            p = jnp.where(wid == w, jnp.int32(_P_LIST[_ROWS_PER_W * w + r]), p)

        c_tm = pltpu.async_copy(tm_hbm.at[pl.ds(i, 1), :], tmbuf, sem)
        c_fm = pltpu.async_copy(fm_hbm.at[pl.ds(i, 1), :], fmbuf, sem)
        c_tm.wait()
        c_fm.wait()

        def group(g, carry):
            lb = i * _F + g * _LPG
            pb = p * _F + g * _LPG
            cps = []
            for j in range(_LPG):
                cps.append(pltpu.async_copy(
                    x_hbm.at[pl.ds(lb + j, 1), :],
                    xbuf.at[pl.ds(j, 1), pl.ds(0, _T)], sem))
                cps.append(pltpu.async_copy(
                    x_hbm.at[pl.ds(pb + j, 1), :],
                    ybuf.at[pl.ds(j, 1), pl.ds(0, _T)], sem))
            for c in cps:
                c.wait()

            for j in range(_LPG):
                l = g * _LPG + j
                fms = plsc.load_gather(
                    fmbuf,
                    [jnp.zeros((_LANES,), jnp.int32),
                     jnp.full((_LANES,), l, jnp.int32)])

                def tstep(t, _, j=j, fms=fms):
                    for u in range(_UNROLL):
                        sl = pl.ds((t * _UNROLL + u) * _LANES, _LANES)
                        xv = xbuf[j, sl]
                        yv = ybuf[j, sl]
                        mv = jnp.maximum(fms, tmbuf[0, sl])
                        blend = 0.5 * (xv + yv)
                        xbuf[j, sl] = jnp.where(mv > 0.0, blend, xv)
                    return 0

                lax.fori_loop(0, _NV // _UNROLL, tstep, 0)

            ocps = []
            for j in range(_LPG):
                ocps.append(pltpu.async_copy(
                    xbuf.at[pl.ds(j, 1), pl.ds(0, _T)],
                    out_hbm.at[pl.ds(lb + j, 1), :], osem))
            for c in ocps:
                c.wait()
            return carry

        lax.fori_loop(0, _NG, group, 0)


_sc_blend = functools.partial(
    pl.kernel,
    out_type=jax.ShapeDtypeStruct((_B * _F, _T), jnp.float32),
    mesh=plsc.VectorSubcoreMesh(core_axis_name="c", subcore_axis_name="s"),
    scratch_types=[
        pltpu.VMEM((_LPG, _TP), jnp.float32),
        pltpu.VMEM((_LPG, _TP), jnp.float32),
        pltpu.VMEM((1, _TP), jnp.float32),
        pltpu.VMEM((1, _F), jnp.float32),
        pltpu.SemaphoreType.DMA,
        pltpu.SemaphoreType.DMA,
    ],
    compiler_params=pltpu.CompilerParams(
        use_tc_tiling_on_sc=False, needs_layout_passes=False),
)(_sc_body)


def kernel(x):
    B, C, F, T = x.shape
    fm32 = np.zeros((_B, _F), dtype=np.float32)
    fm32[_FMASK] = 1.0
    tm32 = np.zeros((_B, _TP), dtype=np.float32)
    tm32[:, :_T][_TMASK] = 1.0

    xr = x.reshape(_B * _F, _T)
    out = _sc_blend(xr, jnp.asarray(fm32), jnp.asarray(tm32))
    aug = out.reshape(B, C, F, T)
    return (aug,
            jnp.asarray(_FMASK),
            jnp.asarray(_TMASK),
            jnp.asarray(_PARTNER, dtype=jnp.int64))


# speedup vs baseline: 1.1729x; 1.1729x over previous
import functools

import numpy as np
import jax
import jax.numpy as jnp
from jax import lax
from jax.experimental import pallas as pl
from jax.experimental.pallas import tpu as pltpu
from jax.experimental.pallas import tpu_sc as plsc

_FREQ_MASK_PARAM = 27
_TIME_MASK_PARAM = 100
_NUM_FREQ_MASKS = 2
_NUM_TIME_MASKS = 2

_B, _F, _T = 64, 128, 3000
_TP = 3008
_LPG = 8
_NG = _F // _LPG
_NW = 32
_ROWS_PER_W = _B // _NW
_LANES = 16
_UNROLL = 4
_NV = _TP // _LANES


def _mask_consts(B, F, T):
    rng = np.random.RandomState(0)
    partner = np.empty(B, dtype=np.int64)
    for i in range(B):
        j = int(rng.randint(0, B - 1))
        if j >= i:
            j += 1
        partner[i] = j
    fmask = np.zeros((B, F), dtype=bool)
    tmask = np.zeros((B, T), dtype=bool)
    for i in range(B):
        for _ in range(_NUM_FREQ_MASKS):
            f = int(rng.randint(0, _FREQ_MASK_PARAM + 1))
            if f == 0:
                continue
            f0 = int(rng.randint(0, F - f + 1))
            fmask[i, f0:f0 + f] = True
        for _ in range(_NUM_TIME_MASKS):
            t = int(rng.randint(0, _TIME_MASK_PARAM + 1))
            if t == 0:
                continue
            t0 = int(rng.randint(0, T - t + 1))
            tmask[i, t0:t0 + t] = True
    return partner, fmask, tmask


_PARTNER, _FMASK, _TMASK = _mask_consts(_B, _F, _T)
_P_LIST = [int(v) for v in _PARTNER]


def _sc_body(x_hbm, fm_hbm, tm_hbm, out_hbm,
             xb0, xb1, yb0, yb1, tmbuf, fmbuf,
             msem, is0, is1, os0, os1):
    nc = 2
    wid = lax.axis_index("s") * nc + lax.axis_index("c")
    xbufs, ybufs = (xb0, xb1), (yb0, yb1)
    isems, osems = (is0, is1), (os0, os1)

    for r in range(_ROWS_PER_W):
        i = wid * _ROWS_PER_W + r
        p = jnp.int32(_P_LIST[r])
        for w in range(1, _NW):
            p = jnp.where(wid == w, jnp.int32(_P_LIST[_ROWS_PER_W * w + r]), p)

        c_tm = pltpu.async_copy(tm_hbm.at[pl.ds(i, 1), :], tmbuf, msem)
        c_fm = pltpu.async_copy(fm_hbm.at[pl.ds(i, 1), :], fmbuf, msem)
        c_tm.wait()
        c_fm.wait()

        def issue_in(g, s):
            lb = i * _F + g * _LPG
            pb = p * _F + g * _LPG
            pltpu.async_copy(x_hbm.at[pl.ds(lb, _LPG), :],
                             xbufs[s].at[:, pl.ds(0, _T)], isems[s])
            pltpu.async_copy(x_hbm.at[pl.ds(pb, _LPG), :],
                             ybufs[s].at[:, pl.ds(0, _T)], isems[s])

        def wait_in(s):
            pltpu.make_async_copy(x_hbm.at[pl.ds(0, _LPG), :],
                                  xbufs[s].at[:, pl.ds(0, _T)], isems[s]).wait()
            pltpu.make_async_copy(x_hbm.at[pl.ds(0, _LPG), :],
                                  ybufs[s].at[:, pl.ds(0, _T)], isems[s]).wait()

        def issue_out(g, s):
            lb = i * _F + g * _LPG
            pltpu.async_copy(xbufs[s].at[:, pl.ds(0, _T)],
                             out_hbm.at[pl.ds(lb, _LPG), :], osems[s])

        def wait_out(s):
            pltpu.make_async_copy(xbufs[s].at[:, pl.ds(0, _T)],
                                  out_hbm.at[pl.ds(0, _LPG), :], osems[s]).wait()

        def compute(g, s):
            xb, yb = xbufs[s], ybufs[s]
            for j in range(_LPG):
                l = g * _LPG + j
                fms = plsc.load_gather(
                    fmbuf,
                    [jnp.zeros((_LANES,), jnp.int32),
                     jnp.full((_LANES,), l, jnp.int32)])

                def tstep(t, _, j=j, fms=fms, xb=xb, yb=yb):
                    for u in range(_UNROLL):
                        sl = pl.ds((t * _UNROLL + u) * _LANES, _LANES)
                        xv = xb[j, sl]
                        yv = yb[j, sl]
                        mv = jnp.maximum(fms, tmbuf[0, sl])
                        blend = 0.5 * (xv + yv)
                        xb[j, sl] = jnp.where(mv > 0.0, blend, xv)
                    return 0

                lax.fori_loop(0, _NV // _UNROLL, tstep, 0)

        issue_in(0, 0)

        def gg_body(gg, carry):
            g0 = 2 * gg
            wait_in(0)

            @pl.when(gg > 0)
            def _():
                wait_out(1)
            issue_in(g0 + 1, 1)
            compute(g0, 0)
            issue_out(g0, 0)

            wait_in(1)

            @pl.when(gg < _NG // 2 - 1)
            def _():
                wait_out(0)
                issue_in(g0 + 2, 0)
            compute(g0 + 1, 1)
            issue_out(g0 + 1, 1)
            return carry

        lax.fori_loop(0, _NG // 2, gg_body, 0)
        wait_out(0)
        wait_out(1)


_sc_blend = functools.partial(
    pl.kernel,
    out_type=jax.ShapeDtypeStruct((_B * _F, _T), jnp.float32),
    mesh=plsc.VectorSubcoreMesh(core_axis_name="c", subcore_axis_name="s"),
    scratch_types=[
        pltpu.VMEM((_LPG, _TP), jnp.float32),
        pltpu.VMEM((_LPG, _TP), jnp.float32),
        pltpu.VMEM((_LPG, _TP), jnp.float32),
        pltpu.VMEM((_LPG, _TP), jnp.float32),
        pltpu.VMEM((1, _TP), jnp.float32),
        pltpu.VMEM((1, _F), jnp.float32),
        pltpu.SemaphoreType.DMA,
        pltpu.SemaphoreType.DMA,
        pltpu.SemaphoreType.DMA,
        pltpu.SemaphoreType.DMA,
        pltpu.SemaphoreType.DMA,
    ],
    compiler_params=pltpu.CompilerParams(
        use_tc_tiling_on_sc=False, needs_layout_passes=False),
)(_sc_body)


def kernel(x):
    B, C, F, T = x.shape
    fm32 = np.zeros((_B, _F), dtype=np.float32)
    fm32[_FMASK] = 1.0
    tm32 = np.zeros((_B, _TP), dtype=np.float32)
    tm32[:, :_T][_TMASK] = 1.0

    xr = x.reshape(_B * _F, _T)
    out = _sc_blend(xr, jnp.asarray(fm32), jnp.asarray(tm32))
    aug = out.reshape(B, C, F, T)
    return (aug,
            jnp.asarray(_FMASK),
            jnp.asarray(_TMASK),
            jnp.asarray(_PARTNER, dtype=jnp.int64))


# speedup vs baseline: 1.1750x; 1.0019x over previous
import functools

import numpy as np
import jax
import jax.numpy as jnp
from jax import lax
from jax.experimental import pallas as pl
from jax.experimental.pallas import tpu as pltpu
from jax.experimental.pallas import tpu_sc as plsc

_FREQ_MASK_PARAM = 27
_TIME_MASK_PARAM = 100
_NUM_FREQ_MASKS = 2
_NUM_TIME_MASKS = 2

_B, _F, _T = 64, 128, 3000
_TP = 3008
_LPG = 8
_NG = _F // _LPG
_NW = 32
_ROWS_PER_W = _B // _NW
_LANES = 16
_UNROLL = 4
_NV = _TP // _LANES


def _mask_consts(B, F, T):
    rng = np.random.RandomState(0)
    partner = np.empty(B, dtype=np.int64)
    for i in range(B):
        j = int(rng.randint(0, B - 1))
        if j >= i:
            j += 1
        partner[i] = j
    fmask = np.zeros((B, F), dtype=bool)
    tmask = np.zeros((B, T), dtype=bool)
    for i in range(B):
        for _ in range(_NUM_FREQ_MASKS):
            f = int(rng.randint(0, _FREQ_MASK_PARAM + 1))
            if f == 0:
                continue
            f0 = int(rng.randint(0, F - f + 1))
            fmask[i, f0:f0 + f] = True
        for _ in range(_NUM_TIME_MASKS):
            t = int(rng.randint(0, _TIME_MASK_PARAM + 1))
            if t == 0:
                continue
            t0 = int(rng.randint(0, T - t + 1))
            tmask[i, t0:t0 + t] = True
    return partner, fmask, tmask


_PARTNER, _FMASK, _TMASK = _mask_consts(_B, _F, _T)
_P_LIST = [int(v) for v in _PARTNER]


def _sc_body(x_hbm, fm_hbm, tm_hbm, out_hbm,
             xb0, xb1, yb0, yb1, tmbuf, fmbuf,
             msem, is0, is1, os0, os1):
    nc = 2
    wid = lax.axis_index("s") * nc + lax.axis_index("c")
    xbufs, ybufs = (xb0, xb1), (yb0, yb1)
    isems, osems = (is0, is1), (os0, os1)

    for r in range(_ROWS_PER_W):
        i = wid * _ROWS_PER_W + r
        p = jnp.int32(_P_LIST[r])
        for w in range(1, _NW):
            p = jnp.where(wid == w, jnp.int32(_P_LIST[_ROWS_PER_W * w + r]), p)

        c_tm = pltpu.async_copy(tm_hbm.at[pl.ds(i, 1), :], tmbuf, msem)
        c_fm = pltpu.async_copy(fm_hbm.at[pl.ds(i, 1), :], fmbuf, msem)
        c_tm.wait()
        c_fm.wait()

        def issue_in(g, s):
            lb = i * _F + g * _LPG
            pb = p * _F + g * _LPG
            pltpu.async_copy(x_hbm.at[pl.ds(lb, _LPG), :],
                             xbufs[s].at[:, pl.ds(0, _T)], isems[s])
            pltpu.async_copy(x_hbm.at[pl.ds(pb, _LPG), :],
                             ybufs[s].at[:, pl.ds(0, _T)], isems[s])

        def wait_in(s):
            pltpu.make_async_copy(x_hbm.at[pl.ds(0, _LPG), :],
                                  xbufs[s].at[:, pl.ds(0, _T)], isems[s]).wait()
            pltpu.make_async_copy(x_hbm.at[pl.ds(0, _LPG), :],
                                  ybufs[s].at[:, pl.ds(0, _T)], isems[s]).wait()

        def issue_out(g, s):
            lb = i * _F + g * _LPG
            pltpu.async_copy(xbufs[s].at[:, pl.ds(0, _T)],
                             out_hbm.at[pl.ds(lb, _LPG), :], osems[s])

        def wait_out(s):
            pltpu.make_async_copy(xbufs[s].at[:, pl.ds(0, _T)],
                                  out_hbm.at[pl.ds(0, _LPG), :], osems[s]).wait()

        def compute(g, s):
            xb, yb = xbufs[s], ybufs[s]
            for j in range(_LPG):
                l = g * _LPG + j
                fms = plsc.load_gather(
                    fmbuf,
                    [jnp.zeros((_LANES,), jnp.int32),
                     jnp.full((_LANES,), l, jnp.int32)])

                @plsc.parallel_loop(0, _NV * _LANES, _LANES, unroll=_UNROLL)
                def _(t0, j=j, fms=fms, xb=xb, yb=yb):
                    sl = pl.ds(t0, _LANES)
                    xv = xb[j, sl]
                    yv = yb[j, sl]
                    mv = jnp.maximum(fms, tmbuf[0, sl])
                    blend = 0.5 * (xv + yv)
                    xb[j, sl] = jnp.where(mv > 0.0, blend, xv)

        issue_in(0, 0)

        def gg_body(gg, carry):
            g0 = 2 * gg
            wait_in(0)

            @pl.when(gg > 0)
            def _():
                wait_out(1)
            issue_in(g0 + 1, 1)
            compute(g0, 0)
            issue_out(g0, 0)

            wait_in(1)

            @pl.when(gg < _NG // 2 - 1)
            def _():
                wait_out(0)
                issue_in(g0 + 2, 0)
            compute(g0 + 1, 1)
            issue_out(g0 + 1, 1)
            return carry

        lax.fori_loop(0, _NG // 2, gg_body, 0)
        wait_out(0)
        wait_out(1)


_sc_blend = functools.partial(
    pl.kernel,
    out_type=jax.ShapeDtypeStruct((_B * _F, _T), jnp.float32),
    mesh=plsc.VectorSubcoreMesh(core_axis_name="c", subcore_axis_name="s"),
    scratch_types=[
        pltpu.VMEM((_LPG, _TP), jnp.float32),
        pltpu.VMEM((_LPG, _TP), jnp.float32),
        pltpu.VMEM((_LPG, _TP), jnp.float32),
        pltpu.VMEM((_LPG, _TP), jnp.float32),
        pltpu.VMEM((1, _TP), jnp.float32),
        pltpu.VMEM((1, _F), jnp.float32),
        pltpu.SemaphoreType.DMA,
        pltpu.SemaphoreType.DMA,
        pltpu.SemaphoreType.DMA,
        pltpu.SemaphoreType.DMA,
        pltpu.SemaphoreType.DMA,
    ],
    compiler_params=pltpu.CompilerParams(
        use_tc_tiling_on_sc=False, needs_layout_passes=False),
)(_sc_body)


def kernel(x):
    B, C, F, T = x.shape
    fm32 = np.zeros((_B, _F), dtype=np.float32)
    fm32[_FMASK] = 1.0
    tm32 = np.zeros((_B, _TP), dtype=np.float32)
    tm32[:, :_T][_TMASK] = 1.0

    xr = x.reshape(_B * _F, _T)
    out = _sc_blend(xr, jnp.asarray(fm32), jnp.asarray(tm32))
    aug = out.reshape(B, C, F, T)
    return (aug,
            jnp.asarray(_FMASK),
            jnp.asarray(_TMASK),
            jnp.asarray(_PARTNER, dtype=jnp.int64))
